# baseline (speedup 1.0000x reference)
import functools

import jax
import jax.numpy as jnp
from jax import lax
from jax.experimental import pallas as pl
from jax.experimental.pallas import tpu as pltpu
from jax.experimental.pallas import tpu_sc as plsc

B, S, H = 1, 2048, 1024
T = B * S
M = 1408
MS = 2816
E = 8
TOPK = 2

P = T * TOPK
TM = 256
PT = P + E * TM
NT = PT // TM

TT = 256
NTT = T // TT

NW = 32
PAIRS_W = P // NW
CH = 64
CHD = 128

HP = H // 2


def _pack_bf16(xf):
    lo = lax.bitcast_convert_type(xf[:, :HP].astype(jnp.bfloat16),
                                  jnp.uint16).astype(jnp.int32)
    hi = lax.bitcast_convert_type(xf[:, HP:].astype(jnp.bfloat16),
                                  jnp.uint16).astype(jnp.int32)
    return lo | (hi << 16)


def _unpack_bf16(xi):
    flo = lax.bitcast_convert_type(xi << 16, jnp.float32)
    fhi = lax.bitcast_convert_type(xi & jnp.int32(-65536), jnp.float32)
    return jnp.concatenate([flo, fhi], axis=1)


def _router_a_body(flat_ref, rw_ref, w0_ref, w1_ref, fb_ref, i0_ref, i1_ref):
    x = flat_ref[...]
    fb_ref[...] = _pack_bf16(x)
    logits = jnp.dot(x, rw_ref[...], preferred_element_type=jnp.float32)
    lane = lax.broadcasted_iota(jnp.int32, (TT, E), 1)
    cols = []
    for e in range(E):
        le = logits[:, e:e + 1]
        gt = jnp.sum((logits > le).astype(jnp.float32), axis=1, keepdims=True)
        eq_lt = jnp.sum(((logits == le) & (lane < e)).astype(jnp.float32),
                        axis=1, keepdims=True)
        cols.append(gt + eq_lt)
    rank = jnp.concatenate(cols, axis=1)
    ind0 = (rank == 0.0).astype(jnp.float32)
    ind1 = (rank == 1.0).astype(jnp.float32)
    mx = jnp.max(logits, axis=1, keepdims=True)
    ex = jnp.exp(logits - mx) * (ind0 + ind1)
    w_full = ex / jnp.sum(ex, axis=1, keepdims=True)
    w0_ref[...] = jnp.sum(w_full * ind0, axis=1, keepdims=True)
    w1_ref[...] = jnp.sum(w_full * ind1, axis=1, keepdims=True)
    i0_ref[...] = ind0
    i1_ref[...] = ind1


def _router_b_body(i0_ref, i1_ref, pos_ref, st_ref):
    sizes = (jnp.sum(i0_ref[...], axis=0, keepdims=True)
             + jnp.sum(i1_ref[...], axis=0, keepdims=True))
    padded = jnp.ceil(sizes / float(TM)) * float(TM)
    r8 = lax.broadcasted_iota(jnp.int32, (E, E), 0)
    c8 = lax.broadcasted_iota(jnp.int32, (E, E), 1)
    u_strict = (r8 < c8).astype(jnp.float32)
    offs = jnp.dot(padded, u_strict, preferred_element_type=jnp.float32)

    r128 = lax.broadcasted_iota(jnp.int32, (128, 128), 0)
    c128 = lax.broadcasted_iota(jnp.int32, (128, 128), 1)
    l_strict = (c128 < r128).astype(jnp.float32)

    def make_body(ind_ref, pos_base):
        def body(b, carry):
            indb = ind_ref[pl.ds(b * 128, 128), :]
            cum = jnp.dot(l_strict, indb, preferred_element_type=jnp.float32)
            posb = jnp.sum(indb * (cum + carry + offs), axis=1, keepdims=True)
            pos_ref[pl.ds(pos_base + b * 128, 128), :] = posb.astype(jnp.int32)
            return carry + jnp.sum(indb, axis=0, keepdims=True)
        return body

    carry = lax.fori_loop(0, T // 128, make_body(i0_ref, 0),
                          jnp.zeros((1, E), jnp.float32))
    lax.fori_loop(0, T // 128, make_body(i1_ref, T), carry)

    tot = jnp.sum(padded, axis=1, keepdims=True)
    st_row = jnp.concatenate(
        [offs / float(TM), tot / float(TM), jnp.zeros((1, 7), jnp.float32)],
        axis=1)
    st_ref[...] = st_row.astype(jnp.int32)


NCH = 8
HC = H // NCH
MC = M // NCH


def _group_manual(st_ref, x_hbm, wgu_hbm, wd_hbm, y_hbm,
                  xbuf, wgubuf, wdbuf, ybuf, sem_x, sem_w, sem_y):
    def w_copies(e, slot):
        cs = []
        for c in range(NCH):
            cs.append(pltpu.make_async_copy(
                wgu_hbm.at[e, pl.ds(c * HC, HC), :],
                wgubuf.at[slot, pl.ds(c * HC, HC), :], sem_w.at[slot]))
            cs.append(pltpu.make_async_copy(
                wd_hbm.at[e, pl.ds(c * MC, MC), :],
                wdbuf.at[slot, pl.ds(c * MC, MC), :], sem_w.at[slot]))
        return cs

    def x_copy(t, slot):
        return pltpu.make_async_copy(
            x_hbm.at[pl.ds(t * TM, TM), :], xbuf.at[slot], sem_x.at[slot])

    def y_copy(t, slot):
        return pltpu.make_async_copy(
            ybuf.at[slot], y_hbm.at[pl.ds(t * TM, TM), :], sem_y.at[slot])

    total = st_ref[8]
    for cpy in w_copies(0, 0):
        cpy.start()

    @pl.when(total > 0)
    def _():
        x_copy(0, 0).start()

    def make_tile_body(wslot):
        def tile_body(t, carry):
            slot = lax.rem(t, 2)
            x_copy(t, slot).wait()

            @pl.when(t + 1 < total)
            def _():
                x_copy(t + 1, lax.rem(t + 1, 2)).start()

            xb = _unpack_bf16(xbuf[slot]).astype(jnp.bfloat16)
            gu = jnp.dot(xb, wgubuf[wslot].astype(jnp.bfloat16),
                         preferred_element_type=jnp.float32)
            gate = gu[:, :M]
            up = gu[:, M:]
            hid = up * gate * jax.nn.sigmoid(gate)
            y = jnp.dot(hid.astype(jnp.bfloat16),
                        wdbuf[wslot].astype(jnp.bfloat16),
                        preferred_element_type=jnp.float32)

            @pl.when(t >= 2)
            def _():
                y_copy(t - 2, slot).wait()

            ybuf[slot] = _pack_bf16(y)
            y_copy(t, slot).start()
            return carry
        return tile_body

    wslot = 0
    for e in range(E):
        if e + 1 < E:
            for cpy in w_copies(e + 1, 1 - wslot):
                cpy.start()
        for cpy in w_copies(e, wslot):
            cpy.wait()
        lax.fori_loop(st_ref[e], st_ref[e + 1], make_tile_body(wslot), 0)
        wslot = 1 - wslot

    def drain_body(t, carry):
        y_copy(t, lax.rem(t, 2)).wait()
        return carry

    lax.fori_loop(jnp.maximum(total - 2, 0), total, drain_body, 0)


def _final_body(flat_ref, a_ref, b_ref, w0_ref, w1_ref,
                wg_ref, wi_ref, wo_ref, wge_ref, out_ref):
    x = flat_ref[...]
    xb = x.astype(jnp.bfloat16)
    g = jnp.dot(xb, wg_ref[...].astype(jnp.bfloat16),
                preferred_element_type=jnp.float32)
    g = g * jax.nn.sigmoid(g)
    x1 = jnp.dot(xb, wi_ref[...].astype(jnp.bfloat16),
                 preferred_element_type=jnp.float32)
    s = jnp.dot((x1 * g).astype(jnp.bfloat16), wo_ref[...].astype(jnp.bfloat16),
                preferred_element_type=jnp.float32)
    sg = jax.nn.sigmoid(jnp.dot(x, wge_ref[...], preferred_element_type=jnp.float32))
    out_ref[...] = (w0_ref[...] * _unpack_bf16(a_ref[...])
                    + w1_ref[...] * _unpack_bf16(b_ref[...]) + sg * s)


def _sc_mesh():
    return plsc.VectorSubcoreMesh(core_axis_name="c", subcore_axis_name="s")


def _make_dispatch():
    @functools.partial(
        pl.kernel,
        mesh=_sc_mesh(),
        out_type=jax.ShapeDtypeStruct((PT, HP), jnp.int32),
        scratch_types=[
            pltpu.VMEM((CHD, HP), jnp.int32),
            pltpu.VMEM((CHD,), jnp.int32),
            pltpu.SemaphoreType.DMA,
        ],
    )
    def dispatch(flat_hbm, pos_hbm, xpad_hbm, xbuf, idxbuf, sem):
        wid = lax.axis_index("s") * 2 + lax.axis_index("c")
        for c in range(PAIRS_W // CHD):
            g0 = wid * PAIRS_W + c * CHD
            src = lax.rem(g0, T)
            pltpu.sync_copy(flat_hbm.at[pl.ds(src, CHD)], xbuf)
            pltpu.sync_copy(pos_hbm.at[pl.ds(g0, CHD)], idxbuf)
            pltpu.async_copy(xbuf, xpad_hbm.at[idxbuf], sem).wait()

    return dispatch


def _make_combine():
    @functools.partial(
        pl.kernel,
        mesh=_sc_mesh(),
        out_type=[
            jax.ShapeDtypeStruct((T, HP), jnp.int32),
            jax.ShapeDtypeStruct((T, HP), jnp.int32),
        ],
        scratch_types=[
            pltpu.VMEM((CH, HP), jnp.int32),
            pltpu.VMEM((CH,), jnp.int32),
            pltpu.SemaphoreType.DMA,
        ],
    )
    def combine(y_hbm, pos_hbm, a_hbm, b_hbm, buf, idxbuf, sem):
        wid = lax.axis_index("s") * 2 + lax.axis_index("c")
        base = wid * CH
        for c, dst in ((0, a_hbm), (1, b_hbm)):
            pltpu.sync_copy(pos_hbm.at[pl.ds(c * T + base, CH)], idxbuf)
            pltpu.async_copy(y_hbm.at[idxbuf], buf, sem).wait()
            pltpu.sync_copy(buf, dst.at[pl.ds(base, CH)])

    return combine


def kernel(hidden_states, router_w, expert_gate_up, expert_down,
           shared_gate_w, shared_inter_w, shared_out_w, shared_expert_gate_w):
    flat = hidden_states.reshape(T, H)

    w0, w1, flat_bf, ind0, ind1 = pl.pallas_call(
        _router_a_body,
        grid=(NTT,),
        in_specs=[
            pl.BlockSpec((TT, H), lambda t: (t, 0)),
            pl.BlockSpec((H, E), lambda t: (0, 0)),
        ],
        out_specs=[
            pl.BlockSpec((TT, 1), lambda t: (t, 0)),
            pl.BlockSpec((TT, 1), lambda t: (t, 0)),
            pl.BlockSpec((TT, HP), lambda t: (t, 0)),
            pl.BlockSpec((TT, E), lambda t: (t, 0)),
            pl.BlockSpec((TT, E), lambda t: (t, 0)),
        ],
        out_shape=[
            jax.ShapeDtypeStruct((T, 1), jnp.float32),
            jax.ShapeDtypeStruct((T, 1), jnp.float32),
            jax.ShapeDtypeStruct((T, HP), jnp.int32),
            jax.ShapeDtypeStruct((T, E), jnp.float32),
            jax.ShapeDtypeStruct((T, E), jnp.float32),
        ],
    )(flat, router_w)

    pos, st = pl.pallas_call(
        _router_b_body,
        out_shape=[
            jax.ShapeDtypeStruct((P, 1), jnp.int32),
            jax.ShapeDtypeStruct((1, 16), jnp.int32),
        ],
    )(ind0, ind1)
    pos = pos.reshape(P)
    st = st.reshape(16)

    x_pad = _make_dispatch()(flat_bf, pos)

    y = pl.pallas_call(
        _group_manual,
        in_specs=[
            pl.BlockSpec(memory_space=pltpu.SMEM),
            pl.BlockSpec(memory_space=pl.ANY),
            pl.BlockSpec(memory_space=pl.ANY),
            pl.BlockSpec(memory_space=pl.ANY),
        ],
        out_specs=pl.BlockSpec(memory_space=pl.ANY),
        out_shape=jax.ShapeDtypeStruct((PT, HP), jnp.int32),
        scratch_shapes=[
            pltpu.VMEM((2, TM, HP), jnp.int32),
            pltpu.VMEM((2, H, 2 * M), jnp.float32),
            pltpu.VMEM((2, M, H), jnp.float32),
            pltpu.VMEM((2, TM, HP), jnp.int32),
            pltpu.SemaphoreType.DMA((2,)),
            pltpu.SemaphoreType.DMA((2,)),
            pltpu.SemaphoreType.DMA((2,)),
        ],
    )(st, x_pad, expert_gate_up, expert_down)

    a, b = _make_combine()(y, pos)

    out = pl.pallas_call(
        _final_body,
        grid=(NTT,),
        in_specs=[
            pl.BlockSpec((TT, H), lambda t: (t, 0)),
            pl.BlockSpec((TT, HP), lambda t: (t, 0)),
            pl.BlockSpec((TT, HP), lambda t: (t, 0)),
            pl.BlockSpec((TT, 1), lambda t: (t, 0)),
            pl.BlockSpec((TT, 1), lambda t: (t, 0)),
            pl.BlockSpec((H, MS), lambda t: (0, 0)),
            pl.BlockSpec((H, MS), lambda t: (0, 0)),
            pl.BlockSpec((MS, H), lambda t: (0, 0)),
            pl.BlockSpec((H, 1), lambda t: (0, 0)),
        ],
        out_specs=pl.BlockSpec((TT, H), lambda t: (t, 0)),
        out_shape=jax.ShapeDtypeStruct((T, H), jnp.float32),
    )(flat, a, b, w0, w1, shared_gate_w, shared_inter_w, shared_out_w,
      shared_expert_gate_w)

    return out.reshape(B, S, H)


# speedup vs baseline: 1.0157x; 1.0157x over previous
import functools

import jax
import jax.numpy as jnp
from jax import lax
from jax.experimental import pallas as pl
from jax.experimental.pallas import tpu as pltpu
from jax.experimental.pallas import tpu_sc as plsc

B, S, H = 1, 2048, 1024
T = B * S
M = 1408
MS = 2816
E = 8
TOPK = 2

P = T * TOPK
TM = 256
PT = P + E * TM
NT = PT // TM

TT = 256
NTT = T // TT

NW = 32
PAIRS_W = P // NW
CH = 64
CHD = 128

HP = H // 2


def _pack_bf16(xf):
    lo = lax.bitcast_convert_type(xf[:, :HP].astype(jnp.bfloat16),
                                  jnp.uint16).astype(jnp.int32)
    hi = lax.bitcast_convert_type(xf[:, HP:].astype(jnp.bfloat16),
                                  jnp.uint16).astype(jnp.int32)
    return lo | (hi << 16)


def _unpack_bf16(xi):
    flo = lax.bitcast_convert_type(xi << 16, jnp.float32)
    fhi = lax.bitcast_convert_type(xi & jnp.int32(-65536), jnp.float32)
    return jnp.concatenate([flo, fhi], axis=1)


def _router_body(flat_ref, rw_ref, w0_ref, w1_ref, pos_ref, st_ref, fb_ref,
                 ind_ref):
    x = flat_ref[...]
    fb_ref[...] = _pack_bf16(x)
    logits = jnp.dot(x, rw_ref[...], preferred_element_type=jnp.float32)
    lane = lax.broadcasted_iota(jnp.int32, (T, E), 1)
    cols = []
    for e in range(E):
        le = logits[:, e:e + 1]
        gt = jnp.sum((logits > le).astype(jnp.float32), axis=1, keepdims=True)
        eq_lt = jnp.sum(((logits == le) & (lane < e)).astype(jnp.float32),
                        axis=1, keepdims=True)
        cols.append(gt + eq_lt)
    rank = jnp.concatenate(cols, axis=1)
    ind0 = (rank == 0.0).astype(jnp.float32)
    ind1 = (rank == 1.0).astype(jnp.float32)
    mx = jnp.max(logits, axis=1, keepdims=True)
    ex = jnp.exp(logits - mx) * (ind0 + ind1)
    w_full = ex / jnp.sum(ex, axis=1, keepdims=True)
    w0_ref[...] = jnp.sum(w_full * ind0, axis=1, keepdims=True)
    w1_ref[...] = jnp.sum(w_full * ind1, axis=1, keepdims=True)

    ind_ref[0:T, :] = ind0
    ind_ref[T:P, :] = ind1
    sizes = jnp.sum(ind_ref[...], axis=0, keepdims=True)
    padded = jnp.ceil(sizes / float(TM)) * float(TM)
    r8 = lax.broadcasted_iota(jnp.int32, (E, E), 0)
    c8 = lax.broadcasted_iota(jnp.int32, (E, E), 1)
    u_strict = (r8 < c8).astype(jnp.float32)
    offs = jnp.dot(padded, u_strict, preferred_element_type=jnp.float32)

    r128 = lax.broadcasted_iota(jnp.int32, (128, 128), 0)
    c128 = lax.broadcasted_iota(jnp.int32, (128, 128), 1)
    l_strict = (c128 < r128).astype(jnp.float32)

    def body(b, carry):
        indb = ind_ref[pl.ds(b * 128, 128), :]
        cum = jnp.dot(l_strict, indb, preferred_element_type=jnp.float32)
        posb = jnp.sum(indb * (cum + carry + offs), axis=1, keepdims=True)
        pos_ref[pl.ds(b * 128, 128), :] = posb.astype(jnp.int32)
        return carry + jnp.sum(indb, axis=0, keepdims=True)

    lax.fori_loop(0, P // 128, body, jnp.zeros((1, E), jnp.float32))

    tot = jnp.sum(padded, axis=1, keepdims=True)
    st_row = jnp.concatenate(
        [offs / float(TM), tot / float(TM), jnp.zeros((1, 7), jnp.float32)],
        axis=1)
    st_ref[...] = st_row.astype(jnp.int32)


NCH = 8
HC = H // NCH
MC = M // NCH


def _group_manual(st_ref, x_hbm, wgu_hbm, wd_hbm, y_hbm,
                  xbuf, wgubuf, wdbuf, ybuf, sem_x, sem_w, sem_y):
    def w_copies(e, slot):
        cs = []
        for c in range(NCH):
            cs.append(pltpu.make_async_copy(
                wgu_hbm.at[e, pl.ds(c * HC, HC), :],
                wgubuf.at[slot, pl.ds(c * HC, HC), :], sem_w.at[slot]))
            cs.append(pltpu.make_async_copy(
                wd_hbm.at[e, pl.ds(c * MC, MC), :],
                wdbuf.at[slot, pl.ds(c * MC, MC), :], sem_w.at[slot]))
        return cs

    def x_copy(t, slot):
        return pltpu.make_async_copy(
            x_hbm.at[pl.ds(t * TM, TM), :], xbuf.at[slot], sem_x.at[slot])

    def y_copy(t, slot):
        return pltpu.make_async_copy(
            ybuf.at[slot], y_hbm.at[pl.ds(t * TM, TM), :], sem_y.at[slot])

    total = st_ref[8]
    for cpy in w_copies(0, 0):
        cpy.start()

    @pl.when(total > 0)
    def _():
        x_copy(0, 0).start()

    def make_tile_body(wslot):
        def tile_body(t, carry):
            slot = lax.rem(t, 2)
            x_copy(t, slot).wait()

            @pl.when(t + 1 < total)
            def _():
                x_copy(t + 1, lax.rem(t + 1, 2)).start()

            xb = _unpack_bf16(xbuf[slot]).astype(jnp.bfloat16)
            gu = jnp.dot(xb, wgubuf[wslot].astype(jnp.bfloat16),
                         preferred_element_type=jnp.float32)
            gate = gu[:, :M]
            up = gu[:, M:]
            hid = up * gate * jax.nn.sigmoid(gate)
            y = jnp.dot(hid.astype(jnp.bfloat16),
                        wdbuf[wslot].astype(jnp.bfloat16),
                        preferred_element_type=jnp.float32)

            @pl.when(t >= 2)
            def _():
                y_copy(t - 2, slot).wait()

            ybuf[slot] = _pack_bf16(y)
            y_copy(t, slot).start()
            return carry
        return tile_body

    wslot = 0
    for e in range(E):
        if e + 1 < E:
            for cpy in w_copies(e + 1, 1 - wslot):
                cpy.start()
        for cpy in w_copies(e, wslot):
            cpy.wait()
        lax.fori_loop(st_ref[e], st_ref[e + 1], make_tile_body(wslot), 0)
        wslot = 1 - wslot

    def drain_body(t, carry):
        y_copy(t, lax.rem(t, 2)).wait()
        return carry

    lax.fori_loop(jnp.maximum(total - 2, 0), total, drain_body, 0)


def _final_body(flat_ref, a_ref, b_ref, w0_ref, w1_ref,
                wg_ref, wi_ref, wo_ref, wge_ref, out_ref):
    x = flat_ref[...]
    xb = x.astype(jnp.bfloat16)
    g = jnp.dot(xb, wg_ref[...].astype(jnp.bfloat16),
                preferred_element_type=jnp.float32)
    g = g * jax.nn.sigmoid(g)
    x1 = jnp.dot(xb, wi_ref[...].astype(jnp.bfloat16),
                 preferred_element_type=jnp.float32)
    s = jnp.dot((x1 * g).astype(jnp.bfloat16), wo_ref[...].astype(jnp.bfloat16),
                preferred_element_type=jnp.float32)
    sg = jax.nn.sigmoid(jnp.dot(x, wge_ref[...], preferred_element_type=jnp.float32))
    out_ref[...] = (w0_ref[...] * _unpack_bf16(a_ref[...])
                    + w1_ref[...] * _unpack_bf16(b_ref[...]) + sg * s)


def _sc_mesh():
    return plsc.VectorSubcoreMesh(core_axis_name="c", subcore_axis_name="s")


def _make_dispatch():
    @functools.partial(
        pl.kernel,
        mesh=_sc_mesh(),
        out_type=jax.ShapeDtypeStruct((PT, HP), jnp.int32),
        scratch_types=[
            pltpu.VMEM((CHD, HP), jnp.int32),
            pltpu.VMEM((CHD,), jnp.int32),
            pltpu.SemaphoreType.DMA,
        ],
    )
    def dispatch(flat_hbm, pos_hbm, xpad_hbm, xbuf, idxbuf, sem):
        wid = lax.axis_index("s") * 2 + lax.axis_index("c")
        for c in range(PAIRS_W // CHD):
            g0 = wid * PAIRS_W + c * CHD
            src = lax.rem(g0, T)
            pltpu.sync_copy(flat_hbm.at[pl.ds(src, CHD)], xbuf)
            pltpu.sync_copy(pos_hbm.at[pl.ds(g0, CHD)], idxbuf)
            pltpu.async_copy(xbuf, xpad_hbm.at[idxbuf], sem).wait()

    return dispatch


def _make_combine():
    @functools.partial(
        pl.kernel,
        mesh=_sc_mesh(),
        out_type=[
            jax.ShapeDtypeStruct((T, HP), jnp.int32),
            jax.ShapeDtypeStruct((T, HP), jnp.int32),
        ],
        scratch_types=[
            pltpu.VMEM((CH, HP), jnp.int32),
            pltpu.VMEM((CH,), jnp.int32),
            pltpu.SemaphoreType.DMA,
        ],
    )
    def combine(y_hbm, pos_hbm, a_hbm, b_hbm, buf, idxbuf, sem):
        wid = lax.axis_index("s") * 2 + lax.axis_index("c")
        base = wid * CH
        for c, dst in ((0, a_hbm), (1, b_hbm)):
            pltpu.sync_copy(pos_hbm.at[pl.ds(c * T + base, CH)], idxbuf)
            pltpu.async_copy(y_hbm.at[idxbuf], buf, sem).wait()
            pltpu.sync_copy(buf, dst.at[pl.ds(base, CH)])

    return combine


def kernel(hidden_states, router_w, expert_gate_up, expert_down,
           shared_gate_w, shared_inter_w, shared_out_w, shared_expert_gate_w):
    flat = hidden_states.reshape(T, H)

    w0, w1, pos, st, flat_bf = pl.pallas_call(
        _router_body,
        out_shape=[
            jax.ShapeDtypeStruct((T, 1), jnp.float32),
            jax.ShapeDtypeStruct((T, 1), jnp.float32),
            jax.ShapeDtypeStruct((P, 1), jnp.int32),
            jax.ShapeDtypeStruct((1, 16), jnp.int32),
            jax.ShapeDtypeStruct((T, HP), jnp.int32),
        ],
        scratch_shapes=[pltpu.VMEM((P, E), jnp.float32)],
    )(flat, router_w)
    pos = pos.reshape(P)
    st = st.reshape(16)

    x_pad = _make_dispatch()(flat_bf, pos)

    y = pl.pallas_call(
        _group_manual,
        in_specs=[
            pl.BlockSpec(memory_space=pltpu.SMEM),
            pl.BlockSpec(memory_space=pl.ANY),
            pl.BlockSpec(memory_space=pl.ANY),
            pl.BlockSpec(memory_space=pl.ANY),
        ],
        out_specs=pl.BlockSpec(memory_space=pl.ANY),
        out_shape=jax.ShapeDtypeStruct((PT, HP), jnp.int32),
        scratch_shapes=[
            pltpu.VMEM((2, TM, HP), jnp.int32),
            pltpu.VMEM((2, H, 2 * M), jnp.float32),
            pltpu.VMEM((2, M, H), jnp.float32),
            pltpu.VMEM((2, TM, HP), jnp.int32),
            pltpu.SemaphoreType.DMA((2,)),
            pltpu.SemaphoreType.DMA((2,)),
            pltpu.SemaphoreType.DMA((2,)),
        ],
    )(st, x_pad, expert_gate_up, expert_down)

    a, b = _make_combine()(y, pos)

    out = pl.pallas_call(
        _final_body,
        grid=(NTT,),
        in_specs=[
            pl.BlockSpec((TT, H), lambda t: (t, 0)),
            pl.BlockSpec((TT, HP), lambda t: (t, 0)),
            pl.BlockSpec((TT, HP), lambda t: (t, 0)),
            pl.BlockSpec((TT, 1), lambda t: (t, 0)),
            pl.BlockSpec((TT, 1), lambda t: (t, 0)),
            pl.BlockSpec((H, MS), lambda t: (0, 0)),
            pl.BlockSpec((H, MS), lambda t: (0, 0)),
            pl.BlockSpec((MS, H), lambda t: (0, 0)),
            pl.BlockSpec((H, 1), lambda t: (0, 0)),
        ],
        out_specs=pl.BlockSpec((TT, H), lambda t: (t, 0)),
        out_shape=jax.ShapeDtypeStruct((T, H), jnp.float32),
    )(flat, a, b, w0, w1, shared_gate_w, shared_inter_w, shared_out_w,
      shared_expert_gate_w)

    return out.reshape(B, S, H)
